# baseline (device time: 62074 ns/iter reference)
import functools

import jax
import jax.numpy as jnp
from jax import lax
from jax.experimental import pallas as pl
from jax.experimental.pallas import tpu as pltpu

N_DEV = 16
M = 1024
CHUNK = M // N_DEV
NGRP = 8
CPG = N_DEV // NGRP
GROWS = M // NGRP


def kernel(x, W1, W2):
    m, k = x.shape
    _, h_per = W1.shape
    _, n = W2.shape

    def body(x_ref, w1_ref, w2_ref, out_ref,
             stage, rs_buf, ag_stage, send_sems, recv_sems, ag_sems):
        my = lax.axis_index("i")

        def peers():
            for o in range(1, N_DEV):
                yield lax.rem(my + o, N_DEV)

        barrier_sem = pltpu.get_barrier_semaphore()
        for p in peers():
            pl.semaphore_signal(
                barrier_sem, inc=1,
                device_id=(p,), device_id_type=pl.DeviceIdType.MESH,
            )

        w1b = w1_ref[...].astype(jnp.bfloat16)
        w2b = w2_ref[...].astype(jnp.bfloat16)

        for i in range(NGRP):
            g = lax.rem(my // CPG + 1 + i, NGRP)
            xg = x_ref[pl.ds(g * GROWS, GROWS), :].astype(jnp.bfloat16)
            hg = jnp.dot(xg, w1b, preferred_element_type=jnp.float32)
            hgb = jnp.maximum(hg, 0.0).astype(jnp.bfloat16)
            pg = jnp.dot(hgb, w2b, preferred_element_type=jnp.float32)
            stage[pl.ds(CPG * g, CPG)] = (
                pg.astype(jnp.bfloat16).reshape(CPG, CHUNK, n)
            )
            if i == 0:
                pl.semaphore_wait(barrier_sem, N_DEV - 1)
            for j in range(CPG):
                c = CPG * g + lax.rem(my + j, CPG)

                @pl.when(c != my)
                def _(c=c):
                    pltpu.make_async_remote_copy(
                        src_ref=stage.at[c],
                        dst_ref=rs_buf.at[my],
                        send_sem=send_sems.at[c],
                        recv_sem=recv_sems.at[my],
                        device_id=(c,),
                        device_id_type=pl.DeviceIdType.MESH,
                    ).start()

        red = stage[my].astype(jnp.float32)
        for p in peers():
            pltpu.make_async_remote_copy(
                src_ref=rs_buf.at[p], dst_ref=rs_buf.at[p],
                send_sem=send_sems.at[p], recv_sem=recv_sems.at[p],
                device_id=(p,), device_id_type=pl.DeviceIdType.MESH,
            ).wait_recv()
            red = red + rs_buf[p].astype(jnp.float32)
        ag_stage[...] = red.astype(jnp.bfloat16)
        for p in peers():
            pltpu.make_async_remote_copy(
                src_ref=stage.at[p], dst_ref=rs_buf.at[my],
                send_sem=send_sems.at[p], recv_sem=recv_sems.at[my],
                device_id=(p,), device_id_type=pl.DeviceIdType.MESH,
            ).wait_send()

        for p in peers():
            pltpu.make_async_remote_copy(
                src_ref=ag_stage,
                dst_ref=out_ref.at[pl.ds(my * CHUNK, CHUNK), :],
                send_sem=send_sems.at[p],
                recv_sem=ag_sems.at[my],
                device_id=(p,),
                device_id_type=pl.DeviceIdType.MESH,
            ).start()
        out_ref[pl.ds(my * CHUNK, CHUNK), :] = ag_stage[...]
        for p in peers():
            pltpu.make_async_remote_copy(
                src_ref=ag_stage,
                dst_ref=out_ref.at[pl.ds(p * CHUNK, CHUNK), :],
                send_sem=send_sems.at[p], recv_sem=ag_sems.at[p],
                device_id=(p,), device_id_type=pl.DeviceIdType.MESH,
            ).wait_recv()
        for p in peers():
            pltpu.make_async_remote_copy(
                src_ref=ag_stage,
                dst_ref=out_ref.at[pl.ds(my * CHUNK, CHUNK), :],
                send_sem=send_sems.at[p], recv_sem=ag_sems.at[my],
                device_id=(p,), device_id_type=pl.DeviceIdType.MESH,
            ).wait_send()

    return pl.pallas_call(
        body,
        out_shape=jax.ShapeDtypeStruct((m, n), jnp.bfloat16),
        in_specs=[
            pl.BlockSpec(memory_space=pltpu.VMEM),
            pl.BlockSpec(memory_space=pltpu.VMEM),
            pl.BlockSpec(memory_space=pltpu.VMEM),
        ],
        out_specs=pl.BlockSpec(memory_space=pltpu.VMEM),
        scratch_shapes=[
            pltpu.VMEM((N_DEV, CHUNK, n), jnp.bfloat16),
            pltpu.VMEM((N_DEV, CHUNK, n), jnp.bfloat16),
            pltpu.VMEM((CHUNK, n), jnp.bfloat16),
            pltpu.SemaphoreType.DMA((N_DEV,)),
            pltpu.SemaphoreType.DMA((N_DEV,)),
            pltpu.SemaphoreType.DMA((N_DEV,)),
        ],
        compiler_params=pltpu.CompilerParams(collective_id=0),
    )(x, W1, W2)


# device time: 59898 ns/iter; 1.0363x vs baseline; 1.0363x over previous
import functools

import jax
import jax.numpy as jnp
from jax import lax
from jax.experimental import pallas as pl
from jax.experimental.pallas import tpu as pltpu

N_DEV = 16
M = 1024
CHUNK = M // N_DEV
NQ = 4
QROWS = M // NQ


def kernel(x, W1, W2):
    m, k = x.shape
    _, h_per = W1.shape
    _, n = W2.shape

    def body(x_ref, w1_ref, w2_ref, out_ref,
             stage, rs_buf, ag_stage, send_sems, recv_sems, ag_sems):
        my = lax.axis_index("i")

        def peers():
            for o in range(1, N_DEV):
                yield lax.rem(my + o, N_DEV)

        barrier_sem = pltpu.get_barrier_semaphore()
        for p in peers():
            pl.semaphore_signal(
                barrier_sem, inc=1,
                device_id=(p,), device_id_type=pl.DeviceIdType.MESH,
            )

        w1b = w1_ref[...].astype(jnp.bfloat16)
        w2b = w2_ref[...].astype(jnp.bfloat16)

        for i in range(NQ):
            q = lax.rem(my // NQ + 1 + i, NQ)
            xq = x_ref[pl.ds(q * QROWS, QROWS), :].astype(jnp.bfloat16)
            hq = jnp.dot(xq, w1b, preferred_element_type=jnp.float32)
            hqb = jnp.maximum(hq, 0.0).astype(jnp.bfloat16)
            pq = jnp.dot(hqb, w2b, preferred_element_type=jnp.float32)
            stage[pl.ds(NQ * q, NQ)] = (
                pq.astype(jnp.bfloat16).reshape(NQ, CHUNK, n)
            )
            if i == 0:
                pl.semaphore_wait(barrier_sem, N_DEV - 1)
            for j in range(NQ):
                c = NQ * q + lax.rem(my + j, NQ)

                @pl.when(c != my)
                def _(c=c):
                    pltpu.make_async_remote_copy(
                        src_ref=stage.at[c],
                        dst_ref=rs_buf.at[my],
                        send_sem=send_sems.at[c],
                        recv_sem=recv_sems.at[my],
                        device_id=(c,),
                        device_id_type=pl.DeviceIdType.MESH,
                    ).start()

        red = stage[my].astype(jnp.float32)
        for p in peers():
            pltpu.make_async_remote_copy(
                src_ref=rs_buf.at[p], dst_ref=rs_buf.at[p],
                send_sem=send_sems.at[p], recv_sem=recv_sems.at[p],
                device_id=(p,), device_id_type=pl.DeviceIdType.MESH,
            ).wait_recv()
            red = red + rs_buf[p].astype(jnp.float32)
        ag_stage[...] = red.astype(jnp.bfloat16)
        for p in peers():
            pltpu.make_async_remote_copy(
                src_ref=stage.at[p], dst_ref=rs_buf.at[my],
                send_sem=send_sems.at[p], recv_sem=recv_sems.at[my],
                device_id=(p,), device_id_type=pl.DeviceIdType.MESH,
            ).wait_send()

        for p in peers():
            pltpu.make_async_remote_copy(
                src_ref=ag_stage,
                dst_ref=out_ref.at[pl.ds(my * CHUNK, CHUNK), :],
                send_sem=send_sems.at[p],
                recv_sem=ag_sems.at[my],
                device_id=(p,),
                device_id_type=pl.DeviceIdType.MESH,
            ).start()
        out_ref[pl.ds(my * CHUNK, CHUNK), :] = ag_stage[...]
        for p in peers():
            pltpu.make_async_remote_copy(
                src_ref=ag_stage,
                dst_ref=out_ref.at[pl.ds(p * CHUNK, CHUNK), :],
                send_sem=send_sems.at[p], recv_sem=ag_sems.at[p],
                device_id=(p,), device_id_type=pl.DeviceIdType.MESH,
            ).wait_recv()
        for p in peers():
            pltpu.make_async_remote_copy(
                src_ref=ag_stage,
                dst_ref=out_ref.at[pl.ds(my * CHUNK, CHUNK), :],
                send_sem=send_sems.at[p], recv_sem=ag_sems.at[my],
                device_id=(p,), device_id_type=pl.DeviceIdType.MESH,
            ).wait_send()

    return pl.pallas_call(
        body,
        out_shape=jax.ShapeDtypeStruct((m, n), jnp.bfloat16),
        in_specs=[
            pl.BlockSpec(memory_space=pltpu.VMEM),
            pl.BlockSpec(memory_space=pltpu.VMEM),
            pl.BlockSpec(memory_space=pltpu.VMEM),
        ],
        out_specs=pl.BlockSpec(memory_space=pltpu.VMEM),
        scratch_shapes=[
            pltpu.VMEM((N_DEV, CHUNK, n), jnp.bfloat16),
            pltpu.VMEM((N_DEV, CHUNK, n), jnp.bfloat16),
            pltpu.VMEM((CHUNK, n), jnp.bfloat16),
            pltpu.SemaphoreType.DMA((N_DEV,)),
            pltpu.SemaphoreType.DMA((N_DEV,)),
            pltpu.SemaphoreType.DMA((N_DEV,)),
        ],
        compiler_params=pltpu.CompilerParams(collective_id=0),
    )(x, W1, W2)
